# baseline (speedup 1.0000x reference)
import functools

import jax
import jax.numpy as jnp
from jax import lax
from jax.experimental import pallas as pl
from jax.experimental.pallas import tpu as pltpu
from jax.experimental.pallas import tpu_sc as plsc

EMBED_DIM = 64
G = 128
LOG2G = 7
W = 400
NSLOT = 4


@functools.cache
def _make_lookup(B, D, n_frozen, n_train, out_shape):
    info = plsc.get_sparse_core_info()
    NC, NS, L = info.num_cores, info.num_subcores, info.num_lanes
    NW = NC * NS
    assert B % (8 * NW) == 0 and D % L == 0
    b_per_w = B // NW
    seq = out_shape[1]
    assert W % seq == 0 and b_per_w % W == 0 and b_per_w // W >= 2
    n_win = b_per_w // W
    bat_w = W // seq
    grp_w = W // L
    assert n_win <= L
    n_max = b_per_w // G + 2
    CP = n_max * G + L
    TOP = n_max * G
    SLAB = W + 1
    mesh = plsc.VectorSubcoreMesh(core_axis_name="c", subcore_axis_name="s")

    @functools.partial(
        pl.kernel,
        mesh=mesh,
        out_type=jax.ShapeDtypeStruct(out_shape, jnp.float32),
        compiler_params=pltpu.CompilerParams(use_tc_tiling_on_sc=False,
                                             needs_layout_passes=False),
        scratch_types=[
            pltpu.VMEM((b_per_w // G, G), jnp.int32),
            pltpu.VMEM((CP,), jnp.int32),
            pltpu.VMEM((L,), jnp.int32),
            pltpu.VMEM((NSLOT, G), jnp.int32),
            pltpu.VMEM((NSLOT, G), jnp.int32),
            pltpu.VMEM((NSLOT, G, D), jnp.float32),
            pltpu.VMEM_SHARED((2 * NS * SLAB, D), jnp.float32),
            pltpu.SemaphoreType.DMA((NSLOT,)),
            pltpu.SemaphoreType.DMA((NSLOT,)),
            pltpu.SemaphoreType.DMA((2 * bat_w,)),
        ],
    )
    def lookup(ids_hbm, frozen_hbm, weight_hbm, out_hbm,
               idx_v, cpos, snap, idx_sc, pos_sc, rows, slab, semg, sems,
               semw):
        cid = lax.axis_index("c")
        sid = lax.axis_index("s")
        wid = sid * NC + cid
        base_bat = wid * (b_per_w // seq)
        rows_per_w = b_per_w // G
        pltpu.sync_copy(ids_hbm.at[pl.ds(wid * rows_per_w, rows_per_w)],
                        idx_v)
        lanes = lax.iota(jnp.int32, L)

        def cbody(i, carry):
            cnt_f, cnt_t, af, at, sn, cb, wi = carry
            v = idx_v[i >> 3, pl.ds((i & 7) * L, L)]
            m = v < n_frozen
            local = i * L + lanes
            _, sv = plsc.sort_key_val(v, local)
            pc = plsc.all_reduce_population_count(m)[0]
            cpos[pl.ds(cnt_f, L)] = sv
            cpos[pl.ds(TOP - cnt_t - L, L)] = sv
            cnt_f = cnt_f + pc
            cnt_t = cnt_t + (L - pc)
            af = jnp.where(pc > 0, sv[0], af)
            at = jnp.where(pc < L, sv[L - 1], at)
            hit = cb == grp_w - 1
            sn = jnp.where(hit & (lanes == wi), cnt_f, sn)
            wi = wi + jnp.where(hit, 1, 0)
            cb = jnp.where(hit, 0, cb + 1)
            return cnt_f, cnt_t, af, at, sn, cb, wi

        z = jnp.int32(0)
        nf, nt, lf, lt, sn, _, _ = lax.fori_loop(
            0, b_per_w // L, cbody,
            (z, z, z, z, jnp.zeros((L,), jnp.int32), z, z))
        snap[pl.ds(0, L)] = sn

        def padfill(start, end, fill):
            fill_v = jnp.broadcast_to(fill, (L,))
            for k in range(G // L):
                pos = start + k * L

                @pl.when(pos < end)
                def _():
                    old = cpos[pl.ds(pos, L)]
                    cpos[pl.ds(pos, L)] = jnp.where(pos + lanes < end,
                                                    fill_v, old)

        nf_pad = ((nf + G - 1) >> LOG2G) * G
        padfill(nf, nf_pad, lf)
        dn = ((TOP - nt) >> LOG2G) * G
        padfill(dn, TOP - nt, lt)

        def win_body(w, carry):
            slab_ix = w & 1
            slab_base = (slab_ix * NS + sid) * SLAB
            fs0 = plsc.load_gather(snap, [jnp.maximum(w - 1, 0) + 0 * lanes])
            fs = jnp.where(w > 0, fs0[0], 0)
            fe = plsc.load_gather(snap, [w + 0 * lanes])[0]
            wlo = w * W
            for k in range(bat_w):
                @pl.when(w >= 2)
                def _(k=k):
                    pltpu.make_async_copy(
                        slab.at[pl.ds(slab_base + k * seq, seq)],
                        out_hbm.at[base_bat + (w - 2) * bat_w + k],
                        semw.at[slab_ix * bat_w + k],
                    ).wait()
            segs = [
                (frozen_hbm, fs, fe, jnp.int32(0), 0),
                (weight_hbm, TOP - (wlo + W - fe), TOP - (wlo - fs), dn,
                 n_frozen),
            ]

            def prep_fire(table, a, b_, lo, sub, c):
                s = jnp.maximum(jnp.minimum(a + c * G, b_ - G), lo)
                for k in range(G // L):
                    local = cpos[pl.ds(s + k * L, L)]
                    lc = jnp.clip(local, 0, b_per_w - 1)
                    iv = plsc.load_gather(idx_v, [lc >> LOG2G, lc & (G - 1)])
                    bound = (n_frozen if sub == 0 else n_train) - 1
                    idx_sc[c, pl.ds(k * L, L)] = jnp.clip(iv - sub, 0, bound)
                    u = local - wlo
                    spi = jnp.where((u >= 0) & (u < W), u, W) + slab_base
                    pos_sc[c, pl.ds(k * L, L)] = spi
                pltpu.make_async_copy(
                    table.at[idx_sc.at[c]], rows.at[c], semg.at[c]
                ).start()

            def scat_wait(c):
                pltpu.make_async_copy(
                    rows.at[c], slab.at[pos_sc.at[c]], sems.at[c]
                ).wait()

            (tab_f, af_, bf_, lof, subf), (tab_t, at_, bt_, lot, subt) = segs
            n_f = bf_ - af_
            n_t = bt_ - at_

            for c in range(NSLOT):
                @pl.when(c * G < n_f)
                def _(c=c):
                    prep_fire(tab_f, af_, bf_, lof, subf, c)
            for c in range(NSLOT):
                @pl.when(c * G < n_f)
                def _(c=c):
                    pltpu.make_async_copy(
                        tab_f.at[idx_sc.at[c]], rows.at[c], semg.at[c]
                    ).wait()
                    pltpu.make_async_copy(
                        rows.at[c], slab.at[pos_sc.at[c]], sems.at[c]
                    ).start()

            for c in range(NSLOT):
                @pl.when(c * G < n_t)
                def _(c=c):
                    @pl.when(c * G < n_f)
                    def _():
                        scat_wait(c)

                    prep_fire(tab_t, at_, bt_, lot, subt, c)
            for c in range(NSLOT):
                @pl.when(c * G < n_t)
                def _(c=c):
                    pltpu.make_async_copy(
                        tab_t.at[idx_sc.at[c]], rows.at[c], semg.at[c]
                    ).wait()
                    pltpu.make_async_copy(
                        rows.at[c], slab.at[pos_sc.at[c]], sems.at[c]
                    ).start()

            for c in range(NSLOT):
                @pl.when((c * G < n_f) | (c * G < n_t))
                def _(c=c):
                    scat_wait(c)

            for k in range(bat_w):
                pltpu.make_async_copy(
                    slab.at[pl.ds(slab_base + k * seq, seq)],
                    out_hbm.at[base_bat + w * bat_w + k],
                    semw.at[slab_ix * bat_w + k],
                ).start()
            return carry

        lax.fori_loop(0, n_win, win_body, 0)

        for w_tail in (n_win - 2, n_win - 1):
            slab_ix = w_tail & 1
            slab_base = (slab_ix * NS + sid) * SLAB
            for k in range(bat_w):
                pltpu.make_async_copy(
                    slab.at[pl.ds(slab_base + k * seq, seq)],
                    out_hbm.at[base_bat + w_tail * bat_w + k],
                    semw.at[slab_ix * bat_w + k],
                ).wait()

    return lookup


def kernel(input, frozen_weight, weight):
    B = input.shape[0] * input.shape[1]
    ids = input.reshape(B // G, G).astype(jnp.int32)
    lookup = _make_lookup(B, EMBED_DIM, frozen_weight.shape[0],
                          weight.shape[0], input.shape + (EMBED_DIM,))
    return lookup(ids, frozen_weight, weight)


# speedup vs baseline: 1.0618x; 1.0618x over previous
import functools

import jax
import jax.numpy as jnp
from jax import lax
from jax.experimental import pallas as pl
from jax.experimental.pallas import tpu as pltpu
from jax.experimental.pallas import tpu_sc as plsc

EMBED_DIM = 64
G = 128
LOG2G = 7
H = 1


@functools.cache
def _make_lookup(B, D, n_frozen, out_shape):
    info = plsc.get_sparse_core_info()
    NC, NS, L = info.num_cores, info.num_subcores, info.num_lanes
    NW = NC * NS
    assert B % (8 * NW) == 0 and D % L == 0
    b_per_w = B // NW
    n_max = b_per_w // G + 2
    NB = max(d for d in range(1, 17) if n_max % d == 0)
    AHEAD = min(NB - 1, 12)
    n_grp = n_max // NB
    CP = n_max * G + L
    mesh = plsc.VectorSubcoreMesh(core_axis_name="c", subcore_axis_name="s")

    @functools.partial(
        pl.kernel,
        mesh=mesh,
        out_type=jax.ShapeDtypeStruct((B, D), jnp.float32),
        compiler_params=pltpu.CompilerParams(use_tc_tiling_on_sc=False,
                                             needs_layout_passes=False),
        scratch_types=[
            pltpu.VMEM((b_per_w // G, G), jnp.int32),
            pltpu.VMEM((CP,), jnp.int32),
            pltpu.VMEM((NB, G), jnp.int32),
            pltpu.VMEM((NB, G), jnp.int32),
            pltpu.VMEM((NB, G, D), jnp.float32),
            pltpu.SemaphoreType.DMA((NB,)),
            pltpu.SemaphoreType.DMA((NB,)),
        ],
    )
    def lookup(ids_hbm, frozen_hbm, weight_hbm, out_hbm,
               idx_v, cpos, idx_sc, pos_sc, rows, semg, sems):
        out2 = out_hbm
        wid = lax.axis_index("s") * NC + lax.axis_index("c")
        base = wid * b_per_w
        rows_per_w = b_per_w // G
        pltpu.sync_copy(ids_hbm.at[pl.ds(wid * rows_per_w, rows_per_w)],
                        idx_v)
        lanes = lax.iota(jnp.int32, L)

        TOP = n_max * G

        def cbody(i, carry):
            cnt_f, cnt_t, af, at = carry
            v = idx_v[i >> 3, pl.ds((i & 7) * L, L)]
            m = v < n_frozen
            local = i * L + lanes
            _, sv = plsc.sort_key_val(v, local)
            pc = plsc.all_reduce_population_count(m)[0]
            cpos[pl.ds(cnt_f, L)] = sv
            cpos[pl.ds(TOP - cnt_t - L, L)] = sv
            af = jnp.where(pc > 0, sv[0], af)
            at = jnp.where(pc < L, sv[L - 1], at)
            return cnt_f + pc, cnt_t + (L - pc), af, at

        nf, nt, lf, lt = lax.fori_loop(
            0, b_per_w // L, cbody,
            (jnp.int32(0), jnp.int32(0), jnp.int32(0), jnp.int32(0)),
            unroll=4)

        def pad(start, end, fill):
            fill_v = jnp.broadcast_to(fill, (L,))
            for k in range(G // L):
                pos = start + k * L

                @pl.when(pos < end)
                def _():
                    old = cpos[pl.ds(pos, L)]
                    cpos[pl.ds(pos, L)] = jnp.where(pos + lanes < end,
                                                    fill_v, old)

        nfc = (nf + G - 1) >> LOG2G
        nf_pad = nfc * G
        pad(nf, nf_pad, lf)
        dn = ((TOP - nt) >> LOG2G) * G
        tcc = (TOP - dn) >> LOG2G
        tc = nfc + tcc
        pad(dn, TOP - nt, lt)

        def prep(jn, b2):
            jn_eff = jnp.minimum(jn, tc - 1)
            is_fn = jn_eff < nfc
            s = jnp.where(is_fn, jn_eff * G, dn + (jn_eff - nfc) * G)
            sub = jnp.where(is_fn, 0, n_frozen)
            for k in range(G // L):
                local = cpos[pl.ds(s + k * L, L)]
                iv = plsc.load_gather(idx_v, [local >> LOG2G,
                                              local & (G - 1)])
                idx_sc[b2, pl.ds(k * L, L)] = iv - sub
                pos_sc[b2, pl.ds(k * L, L)] = local + base

            @pl.when(is_fn)
            def _():
                pltpu.make_async_copy(
                    frozen_hbm.at[idx_sc.at[b2]], rows.at[b2], semg.at[b2]
                ).start()

            @pl.when(jnp.logical_not(is_fn))
            def _():
                pltpu.make_async_copy(
                    weight_hbm.at[idx_sc.at[b2]], rows.at[b2], semg.at[b2]
                ).start()

        for b in range(AHEAD):
            prep(jnp.int32(b), b)

        def grp(g, carry):
            for b in range(NB):
                j = g * NB + b
                pltpu.make_async_copy(
                    frozen_hbm.at[idx_sc.at[b]], rows.at[b], semg.at[b]
                ).wait()
                pltpu.make_async_copy(
                    rows.at[b], out2.at[pos_sc.at[b]], sems.at[b]
                ).start()
                b2 = (b + AHEAD) % NB

                @pl.when(j + AHEAD < n_max)
                def _():
                    @pl.when(j >= NB - AHEAD)
                    def _():
                        pltpu.make_async_copy(
                            rows.at[b2], out2.at[pos_sc.at[b2]],
                            sems.at[b2]
                        ).wait()

                    prep(j + AHEAD, b2)

            return carry

        lax.fori_loop(0, n_grp, grp, 0)

        for b in range(NB):
            pltpu.make_async_copy(
                rows.at[b], out2.at[pos_sc.at[b]], sems.at[b]
            ).wait()

    return lookup


def kernel(input, frozen_weight, weight):
    B = input.shape[0] * input.shape[1]
    ids = input.reshape(B // G, G).astype(jnp.int32)
    lookup = _make_lookup(B, EMBED_DIM, frozen_weight.shape[0], None)
    out = lookup(ids, frozen_weight, weight)
    return out.reshape(input.shape + (EMBED_DIM,))


# speedup vs baseline: 1.0625x; 1.0006x over previous
import functools

import jax
import jax.numpy as jnp
from jax import lax
from jax.experimental import pallas as pl
from jax.experimental.pallas import tpu as pltpu
from jax.experimental.pallas import tpu_sc as plsc

EMBED_DIM = 64
G = 128
LOG2G = 7


@functools.cache
def _make_lookup(B, D, n_frozen):
    info = plsc.get_sparse_core_info()
    NC, NS, L = info.num_cores, info.num_subcores, info.num_lanes
    NW = NC * NS
    assert B % (8 * NW) == 0 and D % L == 0
    b_per_w = B // NW
    n_max = b_per_w // G + 2
    NB = max(d for d in range(1, 17) if n_max % d == 0)
    AHEAD = min(NB - 1, 12)
    n_grp = n_max // NB
    CP = n_max * G + L
    mesh = plsc.VectorSubcoreMesh(core_axis_name="c", subcore_axis_name="s")

    @functools.partial(
        pl.kernel,
        mesh=mesh,
        out_type=jax.ShapeDtypeStruct((B, D), jnp.float32),
        compiler_params=pltpu.CompilerParams(use_tc_tiling_on_sc=False,
                                             needs_layout_passes=False),
        scratch_types=[
            pltpu.VMEM((b_per_w // G, G), jnp.int32),
            pltpu.VMEM((CP,), jnp.int32),
            pltpu.VMEM((NB, G), jnp.int32),
            pltpu.VMEM((NB, G), jnp.int32),
            pltpu.VMEM((NB, G, D), jnp.float32),
            pltpu.SemaphoreType.DMA((NB,)),
            pltpu.SemaphoreType.DMA((NB,)),
        ],
    )
    def lookup(ids_hbm, frozen_hbm, weight_hbm, out_hbm,
               idx_v, cpos, idx_sc, pos_sc, rows, semg, sems):
        wid = lax.axis_index("s") * NC + lax.axis_index("c")
        base = wid * b_per_w
        rows_per_w = b_per_w // G
        pltpu.sync_copy(ids_hbm.at[pl.ds(wid * rows_per_w, rows_per_w)],
                        idx_v)
        lanes = lax.iota(jnp.int32, L)

        TOP = n_max * G

        def cbody(i, carry):
            cnt_f, cnt_t, af, at = carry
            v = idx_v[i >> 3, pl.ds((i & 7) * L, L)]
            m = v < n_frozen
            local = i * L + lanes
            _, sv = plsc.sort_key_val(v, local)
            pc = plsc.all_reduce_population_count(m)[0]
            cpos[pl.ds(cnt_f, L)] = sv
            cpos[pl.ds(TOP - cnt_t - L, L)] = sv
            af = jnp.where(pc > 0, sv[0], af)
            at = jnp.where(pc < L, sv[L - 1], at)
            return cnt_f + pc, cnt_t + (L - pc), af, at

        nf, nt, lf, lt = lax.fori_loop(
            0, b_per_w // L, cbody,
            (jnp.int32(0), jnp.int32(0), jnp.int32(0), jnp.int32(0)),
            unroll=4)

        def pad(start, end, fill):
            fill_v = jnp.broadcast_to(fill, (L,))
            for k in range(G // L):
                pos = start + k * L

                @pl.when(pos < end)
                def _():
                    old = cpos[pl.ds(pos, L)]
                    cpos[pl.ds(pos, L)] = jnp.where(pos + lanes < end,
                                                    fill_v, old)

        nfc = (nf + G - 1) >> LOG2G
        nf_pad = nfc * G
        pad(nf, nf_pad, lf)
        dn = ((TOP - nt) >> LOG2G) * G
        tcc = (TOP - dn) >> LOG2G
        tc = nfc + tcc
        pad(dn, TOP - nt, lt)

        def prep(jn, b2):
            jn_eff = jnp.minimum(jn, tc - 1)
            is_fn = jn_eff < nfc
            s = jnp.where(is_fn, jn_eff * G, dn + (jn_eff - nfc) * G)
            sub = jnp.where(is_fn, 0, n_frozen)
            for k in range(G // L):
                local = cpos[pl.ds(s + k * L, L)]
                iv = plsc.load_gather(idx_v, [local >> LOG2G,
                                              local & (G - 1)])
                idx_sc[b2, pl.ds(k * L, L)] = iv - sub
                pos_sc[b2, pl.ds(k * L, L)] = local + base

            @pl.when(is_fn)
            def _():
                pltpu.make_async_copy(
                    frozen_hbm.at[idx_sc.at[b2]], rows.at[b2], semg.at[b2]
                ).start()

            @pl.when(jnp.logical_not(is_fn))
            def _():
                pltpu.make_async_copy(
                    weight_hbm.at[idx_sc.at[b2]], rows.at[b2], semg.at[b2]
                ).start()

        for b in range(AHEAD):
            prep(jnp.int32(b), b)

        def grp(g, carry):
            for b in range(NB):
                j = g * NB + b
                pltpu.make_async_copy(
                    frozen_hbm.at[idx_sc.at[b]], rows.at[b], semg.at[b]
                ).wait()
                pltpu.make_async_copy(
                    rows.at[b], out_hbm.at[pos_sc.at[b]], sems.at[b]
                ).start()
                b2 = (b + AHEAD) % NB

                @pl.when(j + AHEAD < n_max)
                def _():
                    @pl.when(j >= NB - AHEAD)
                    def _():
                        pltpu.make_async_copy(
                            rows.at[b2], out_hbm.at[pos_sc.at[b2]],
                            sems.at[b2]
                        ).wait()

                    prep(j + AHEAD, b2)

            return carry

        lax.fori_loop(0, n_grp, grp, 0)

        for b in range(NB):
            pltpu.make_async_copy(
                rows.at[b], out_hbm.at[pos_sc.at[b]], sems.at[b]
            ).wait()

    return lookup


def kernel(input, frozen_weight, weight):
    B = input.shape[0] * input.shape[1]
    ids = input.reshape(B // G, G).astype(jnp.int32)
    lookup = _make_lookup(B, EMBED_DIM, frozen_weight.shape[0])
    out = lookup(ids, frozen_weight, weight)
    return out.reshape(input.shape + (EMBED_DIM,))
